# Initial kernel scaffold; baseline (speedup 1.0000x reference)
import jax
import jax.numpy as jnp
from jax.experimental import pallas as pl


def kernel(species, edge_src, edge_dst, distances, species_table, W1, b1, Wf1, bf1, Wf2, bf2, W2, b2, W3, b3):
    raise NotImplementedError("write your pallas kernel here")


# speedup vs baseline: 1.2825x; 1.2825x over previous
import functools

import jax
import jax.numpy as jnp
import numpy as np
from jax import lax
from jax.experimental import pallas as pl
from jax.experimental.pallas import tpu as pltpu
from jax.experimental.pallas import tpu_sc as plsc

N = 50000
E = 800000
DIM = 64
HDIM = DIM // 2
NB = 16
NL = 3
CUTOFF = 5.0

RN = 2000
EBLK = 3200

NCORE = 2
NSUB = 16
CH = 80
EPT = E // NSUB
NCHUNK = EPT // CH
ZCH = 200
NZC = N // ZCH


def _softplus(x):
    return jnp.maximum(x, 0.0) + jnp.log1p(jnp.exp(-jnp.abs(x)))


def _encode_body(sp_ref, tbl_ref, w1_ref, b1_ref, xi_ref, xs_ref):
    sp = sp_ref[0, 0, :]
    oh = (sp[:, None] == lax.broadcasted_iota(jnp.int32, (1, 128), 1)).astype(jnp.float32)
    xi0 = jnp.dot(oh, tbl_ref[...], preferred_element_type=jnp.float32)
    xi_ref[...] = xi0
    y = jnp.dot(xi0, w1_ref[...], preferred_element_type=jnp.float32) + b1_ref[...]
    xs_ref[0] = y[:, :HDIM]
    xs_ref[1] = y[:, HDIM:]


def _encode(species3, tbl128, w1, b1r):
    return pl.pallas_call(
        _encode_body,
        grid=(N // RN,),
        in_specs=[
            pl.BlockSpec((1, 1, RN), lambda i: (i, 0, 0)),
            pl.BlockSpec((128, DIM), lambda i: (0, 0)),
            pl.BlockSpec((DIM, DIM), lambda i: (0, 0)),
            pl.BlockSpec((1, DIM), lambda i: (0, 0)),
        ],
        out_specs=[
            pl.BlockSpec((RN, DIM), lambda i: (i, 0)),
            pl.BlockSpec((2, RN, HDIM), lambda i: (0, i, 0)),
        ],
        out_shape=[
            jax.ShapeDtypeStruct((N, DIM), jnp.float32),
            jax.ShapeDtypeStruct((2, N, HDIM), jnp.float32),
        ],
    )(species3, tbl128, w1, b1r)


def _filters_body(d_ref, wf1_ref, bf1_ref, wf2_ref, bf2_ref, out_ref):
    d = d_ref[0, 0, :]
    mu = lax.broadcasted_iota(jnp.int32, (1, NB), 1).astype(jnp.float32) * (
        CUTOFF / (NB - 1))
    sigma = CUTOFF / NB
    rb = jnp.exp(-((d[:, None] - mu) ** 2) * (1.0 / (2.0 * sigma * sigma)))
    for l in range(NL):
        h = _softplus(jnp.dot(rb, wf1_ref[l], preferred_element_type=jnp.float32) + bf1_ref[l])
        w = jnp.dot(h, wf2_ref[l], preferred_element_type=jnp.float32) + bf2_ref[l]
        out_ref[l, 0] = w[:, :HDIM]
        out_ref[l, 1] = w[:, HDIM:]


def _filters(dist3, wf1, bf1r, wf2, bf2r):
    return pl.pallas_call(
        _filters_body,
        grid=(E // EBLK,),
        in_specs=[
            pl.BlockSpec((1, 1, EBLK), lambda i: (i, 0, 0)),
            pl.BlockSpec((NL, NB, DIM), lambda i: (0, 0, 0)),
            pl.BlockSpec((NL, 1, DIM), lambda i: (0, 0, 0)),
            pl.BlockSpec((NL, DIM, DIM), lambda i: (0, 0, 0)),
            pl.BlockSpec((NL, 1, DIM), lambda i: (0, 0, 0)),
        ],
        out_specs=pl.BlockSpec((NL, 2, EBLK, HDIM), lambda i: (0, 0, i, 0)),
        out_shape=jax.ShapeDtypeStruct((NL, 2, E, HDIM), jnp.float32),
    )(dist3, wf1, bf1r, wf2, bf2r)


def _conv_body(xi2_hbm, w2_hbm, src_hbm, dst_hbm, out_hbm,
               dstv, srcv, idxv, wv, rowsv, zbuf, acc, gsem):
    c = lax.axis_index("c")
    s = lax.axis_index("s")
    cbase = c * N

    def _zz(i, _):
        zbuf[i, pl.ds(0, 16)] = jnp.zeros((16,), jnp.float32)
        zbuf[i, pl.ds(16, 16)] = jnp.zeros((16,), jnp.float32)
        return 0
    lax.fori_loop(0, ZCH, _zz, 0)

    def _zero_chunk(k, _):
        cid = k * NSUB + s

        @pl.when(cid < NZC)
        def _():
            pltpu.sync_copy(zbuf, acc.at[pl.ds(cid * ZCH, ZCH)])
        return 0
    lax.fori_loop(0, (NZC + NSUB - 1) // NSUB, _zero_chunk, 0)
    plsc.subcore_barrier()

    ebase = s * EPT

    def _chunk(k, _):
        e0 = ebase + k * CH
        pltpu.sync_copy(dst_hbm.at[pl.ds(e0, CH)], dstv)
        pltpu.sync_copy(src_hbm.at[pl.ds(e0, CH)], srcv)
        pltpu.sync_copy(w2_hbm.at[c, pl.ds(e0, CH)], wv)
        for j in range(CH // 16):
            idxv[pl.ds(j * 16, 16)] = dstv[pl.ds(j * 16, 16)] + cbase
        pltpu.async_copy(xi2_hbm.at[idxv], rowsv, gsem).wait()

        def _mul(r, _):
            rowsv[r, pl.ds(0, 16)] = rowsv[r, pl.ds(0, 16)] * wv[r, pl.ds(0, 16)]
            rowsv[r, pl.ds(16, 16)] = rowsv[r, pl.ds(16, 16)] * wv[r, pl.ds(16, 16)]
            return 0
        lax.fori_loop(0, CH, _mul, 0)
        pltpu.sync_copy(rowsv, acc.at[srcv], add=True)
        return 0

    lax.fori_loop(0, NCHUNK, _chunk, 0)
    plsc.subcore_barrier()

    def _flush_chunk(k, _):
        cid = k * NSUB + s

        @pl.when(cid < NZC)
        def _():
            pltpu.sync_copy(acc.at[pl.ds(cid * ZCH, ZCH)],
                            out_hbm.at[pl.ds(cbase + cid * ZCH, ZCH)])
        return 0
    lax.fori_loop(0, (NZC + NSUB - 1) // NSUB, _flush_chunk, 0)


def _conv(xi2, w2, src, dst):
    mesh = plsc.VectorSubcoreMesh(core_axis_name="c", subcore_axis_name="s")
    kfn = functools.partial(
        pl.kernel, mesh=mesh,
        compiler_params=pltpu.CompilerParams(use_tc_tiling_on_sc=False),
        out_type=jax.ShapeDtypeStruct((2 * N, HDIM), jnp.float32),
        scratch_types=[
            pltpu.VMEM((CH,), jnp.int32),
            pltpu.VMEM((CH,), jnp.int32),
            pltpu.VMEM((CH,), jnp.int32),
            pltpu.VMEM((CH, HDIM), jnp.float32),
            pltpu.VMEM((CH, HDIM), jnp.float32),
            pltpu.VMEM((ZCH, HDIM), jnp.float32),
            pltpu.VMEM_SHARED((N, HDIM), jnp.float32),
            pltpu.SemaphoreType.DMA,
        ],
    )(_conv_body)
    return kfn(xi2, w2, src, dst)


def _post_body_mid(agg_ref, xip_ref, w2_ref, b2_ref, w3_ref, b3_ref,
                   w1n_ref, b1n_ref, xi_ref, xs_ref):
    a = jnp.concatenate([agg_ref[0], agg_ref[1]], axis=1)
    t = _softplus(jnp.dot(a, w2_ref[...], preferred_element_type=jnp.float32) + b2_ref[...])
    y = jnp.dot(t, w3_ref[...], preferred_element_type=jnp.float32) + b3_ref[...] + xip_ref[...]
    xi_ref[...] = y
    z = jnp.dot(y, w1n_ref[...], preferred_element_type=jnp.float32) + b1n_ref[...]
    xs_ref[0] = z[:, :HDIM]
    xs_ref[1] = z[:, HDIM:]


def _post_body_last(agg_ref, xip_ref, w2_ref, b2_ref, w3_ref, b3_ref, xi_ref):
    a = jnp.concatenate([agg_ref[0], agg_ref[1]], axis=1)
    t = _softplus(jnp.dot(a, w2_ref[...], preferred_element_type=jnp.float32) + b2_ref[...])
    y = jnp.dot(t, w3_ref[...], preferred_element_type=jnp.float32) + b3_ref[...] + xip_ref[...]
    xi_ref[...] = y


_MAT_SPEC = pl.BlockSpec((DIM, DIM), lambda i: (0, 0))
_VEC_SPEC = pl.BlockSpec((1, DIM), lambda i: (0, 0))


def _post_mid(agg3, xi_prev, w2, b2r, w3, b3r, w1n, b1nr):
    return pl.pallas_call(
        _post_body_mid,
        grid=(N // RN,),
        in_specs=[
            pl.BlockSpec((2, RN, HDIM), lambda i: (0, i, 0)),
            pl.BlockSpec((RN, DIM), lambda i: (i, 0)),
            _MAT_SPEC, _VEC_SPEC, _MAT_SPEC, _VEC_SPEC, _MAT_SPEC, _VEC_SPEC,
        ],
        out_specs=[
            pl.BlockSpec((RN, DIM), lambda i: (i, 0)),
            pl.BlockSpec((2, RN, HDIM), lambda i: (0, i, 0)),
        ],
        out_shape=[
            jax.ShapeDtypeStruct((N, DIM), jnp.float32),
            jax.ShapeDtypeStruct((2, N, HDIM), jnp.float32),
        ],
    )(agg3, xi_prev, w2, b2r, w3, b3r, w1n, b1nr)


def _post_last(agg3, xi_prev, w2, b2r, w3, b3r):
    return pl.pallas_call(
        _post_body_last,
        grid=(N // RN,),
        in_specs=[
            pl.BlockSpec((2, RN, HDIM), lambda i: (0, i, 0)),
            pl.BlockSpec((RN, DIM), lambda i: (i, 0)),
            _MAT_SPEC, _VEC_SPEC, _MAT_SPEC, _VEC_SPEC,
        ],
        out_specs=pl.BlockSpec((RN, DIM), lambda i: (i, 0)),
        out_shape=jax.ShapeDtypeStruct((N, DIM), jnp.float32),
    )(agg3, xi_prev, w2, b2r, w3, b3r)


def kernel(species, edge_src, edge_dst, distances, species_table,
           W1, b1, Wf1, bf1, Wf2, bf2, W2, b2, W3, b3):
    species3 = species.astype(jnp.int32).reshape(N // RN, 1, RN)
    src = edge_src.astype(jnp.int32)
    dst = edge_dst.astype(jnp.int32)
    dist3 = distances.reshape(E // EBLK, 1, EBLK)
    tbl128 = jnp.zeros((128, DIM), jnp.float32).at[:100].set(species_table)

    b1r = b1.reshape(NL, 1, DIM)
    bf1r = bf1.reshape(NL, 1, DIM)
    bf2r = bf2.reshape(NL, 1, DIM)
    b2r = b2.reshape(NL, 1, DIM)
    b3r = b3.reshape(NL, 1, DIM)

    w_all = _filters(dist3, Wf1, bf1r, Wf2, bf2r)
    xi_prev, xs = _encode(species3, tbl128, W1[0], b1r[0])

    for l in range(NL):
        agg = _conv(xs.reshape(2 * N, HDIM), w_all[l], src, dst)
        agg3 = agg.reshape(2, N, HDIM)
        if l + 1 < NL:
            xi_prev, xs = _post_mid(agg3, xi_prev, W2[l], b2r[l], W3[l], b3r[l],
                                    W1[l + 1], b1r[l + 1])
        else:
            xi_prev = _post_last(agg3, xi_prev, W2[l], b2r[l], W3[l], b3r[l])
    return xi_prev


# speedup vs baseline: 2.8918x; 2.2547x over previous
import functools

import jax
import jax.numpy as jnp
import numpy as np
from jax import lax
from jax.experimental import pallas as pl
from jax.experimental.pallas import tpu as pltpu
from jax.experimental.pallas import tpu_sc as plsc

N = 50000
E = 800000
DIM = 64
HDIM = DIM // 2
NB = 16
NL = 3
CUTOFF = 5.0

RN = 2000
EBLK = 3200

NCORE = 2
NSUB = 16
CH = 80
NCHT = E // CH
MPT = NCHT // NSUB
NKIT = (MPT + 2) // 3
ZCH = 80
NZC = N // ZCH


def _softplus(x):
    return jnp.maximum(x, 0.0) + jnp.log1p(jnp.exp(-jnp.abs(x)))


def _encode_body(sp_ref, tbl_ref, w1_ref, b1_ref, xi_ref, xs_ref):
    sp = sp_ref[0, 0, :]
    oh = (sp[:, None] == lax.broadcasted_iota(jnp.int32, (1, 128), 1)).astype(jnp.float32)
    xi0 = jnp.dot(oh, tbl_ref[...], preferred_element_type=jnp.float32)
    xi_ref[...] = xi0
    y = jnp.dot(xi0, w1_ref[...], preferred_element_type=jnp.float32) + b1_ref[...]
    xs_ref[0] = y[:, :HDIM]
    xs_ref[1] = y[:, HDIM:]


def _encode(species3, tbl128, w1, b1r):
    return pl.pallas_call(
        _encode_body,
        grid=(N // RN,),
        in_specs=[
            pl.BlockSpec((1, 1, RN), lambda i: (i, 0, 0)),
            pl.BlockSpec((128, DIM), lambda i: (0, 0)),
            pl.BlockSpec((DIM, DIM), lambda i: (0, 0)),
            pl.BlockSpec((1, DIM), lambda i: (0, 0)),
        ],
        out_specs=[
            pl.BlockSpec((RN, DIM), lambda i: (i, 0)),
            pl.BlockSpec((2, RN, HDIM), lambda i: (0, i, 0)),
        ],
        out_shape=[
            jax.ShapeDtypeStruct((N, DIM), jnp.float32),
            jax.ShapeDtypeStruct((2, N, HDIM), jnp.float32),
        ],
    )(species3, tbl128, w1, b1r)


def _filters_body(d_ref, wf1_ref, bf1_ref, wf2_ref, bf2_ref, *out_refs):
    d = d_ref[0, 0, :]
    mu = lax.broadcasted_iota(jnp.int32, (1, NB), 1).astype(jnp.float32) * (
        CUTOFF / (NB - 1))
    sigma = CUTOFF / NB
    rb = jnp.exp(-((d[:, None] - mu) ** 2) * (1.0 / (2.0 * sigma * sigma)))
    for l in range(NL):
        h = _softplus(jnp.dot(rb, wf1_ref[l], preferred_element_type=jnp.float32) + bf1_ref[l])
        w = jnp.dot(h, wf2_ref[l], preferred_element_type=jnp.float32) + bf2_ref[l]
        out_refs[l][0] = w[:, :HDIM]
        out_refs[l][1] = w[:, HDIM:]


def _filters(dist3, wf1, bf1r, wf2, bf2r):
    return pl.pallas_call(
        _filters_body,
        grid=(E // EBLK,),
        in_specs=[
            pl.BlockSpec((1, 1, EBLK), lambda i: (i, 0, 0)),
            pl.BlockSpec((NL, NB, DIM), lambda i: (0, 0, 0)),
            pl.BlockSpec((NL, 1, DIM), lambda i: (0, 0, 0)),
            pl.BlockSpec((NL, DIM, DIM), lambda i: (0, 0, 0)),
            pl.BlockSpec((NL, 1, DIM), lambda i: (0, 0, 0)),
        ],
        out_specs=[pl.BlockSpec((2, EBLK, HDIM), lambda i: (0, i, 0))] * NL,
        out_shape=[jax.ShapeDtypeStruct((2, E, HDIM), jnp.float32)] * NL,
    )(dist3, wf1, bf1r, wf2, bf2r)


def _conv_body(xi2_hbm, w2_hbm, src_hbm, dst_hbm, out_hbm,
               d0, d1, d2, sr0, sr1, sr2, ix0, ix1, ix2,
               w0, w1, w2, r0, r1, r2, zbuf, acc,
               f0, f1, f2, g0, g1, g2, s0, s1, s2):
    c = lax.axis_index("c")
    s = lax.axis_index("s")
    cbase = c * N
    dstv = (d0, d1, d2)
    srcv = (sr0, sr1, sr2)
    idxv = (ix0, ix1, ix2)
    wv = (w0, w1, w2)
    rowsv = (r0, r1, r2)
    fsem = (f0, f1, f2)
    gsem = (g0, g1, g2)
    ssem = (s0, s1, s2)

    def _zz(i, _):
        zbuf[i, pl.ds(0, 16)] = jnp.zeros((16,), jnp.float32)
        zbuf[i, pl.ds(16, 16)] = jnp.zeros((16,), jnp.float32)
        return 0
    lax.fori_loop(0, ZCH, _zz, 0)

    def _zero_chunk(k, _):
        cid = k * NSUB + s

        @pl.when(cid < NZC)
        def _():
            pltpu.sync_copy(zbuf, acc.at[pl.ds(cid * ZCH, ZCH)])
        return 0
    lax.fori_loop(0, (NZC + NSUB - 1) // NSUB, _zero_chunk, 0)
    plsc.subcore_barrier()

    def _e0(m):
        return (s + NSUB * m) * CH

    def _valid(m):
        return m < MPT

    def _fetch_copies(j, m):
        e0 = _e0(m)
        return (
            pltpu.make_async_copy(dst_hbm.at[pl.ds(e0, CH)], dstv[j], fsem[j]),
            pltpu.make_async_copy(src_hbm.at[pl.ds(e0, CH)], srcv[j], fsem[j]),
            pltpu.make_async_copy(w2_hbm.at[c, pl.ds(e0, CH)], wv[j], fsem[j]),
        )

    def _fetch_issue(j, m):
        for cp in _fetch_copies(j, m):
            cp.start()

    def _fetch_wait(j, m):
        for cp in _fetch_copies(j, m):
            cp.wait()

    def _idx_compute(j):
        for q in range(CH // 16):
            idxv[j][pl.ds(q * 16, 16)] = dstv[j][pl.ds(q * 16, 16)] + cbase

    def _gather(j):
        return pltpu.make_async_copy(xi2_hbm.at[idxv[j]], rowsv[j], gsem[j])

    def _scatter(j):
        return pltpu.make_async_copy(rowsv[j], acc.at[srcv[j]], ssem[j])

    def _mul(j):
        @plsc.parallel_loop(0, CH, unroll=4)
        def _(r):
            rowsv[j][r, pl.ds(0, 16)] = rowsv[j][r, pl.ds(0, 16)] * wv[j][r, pl.ds(0, 16)]
            rowsv[j][r, pl.ds(16, 16)] = rowsv[j][r, pl.ds(16, 16)] * wv[j][r, pl.ds(16, 16)]

    _fetch_issue(0, 0)
    _fetch_wait(0, 0)
    _idx_compute(0)
    _gather(0).start()
    _fetch_issue(1, 1)

    def _step(k, _):
        for jj in range(3):
            m = 3 * k + jj
            jn = (jj + 1) % 3
            jp = (jj + 2) % 3

            @pl.when(_valid(m + 1))
            def _():
                _fetch_wait(jn, m + 1)
                _idx_compute(jn)
                _gather(jn).start()

            @pl.when(_valid(m))
            def _():
                _gather(jj).wait()
                _mul(jj)
                _scatter(jj).start(add=True)

            @pl.when((m >= 1) & _valid(m - 1))
            def _():
                _scatter(jp).wait()

            @pl.when(_valid(m + 2))
            def _():
                _fetch_issue(jp, m + 2)
        return 0

    lax.fori_loop(0, NKIT, _step, 0)
    plsc.subcore_barrier()

    def _flush_chunk(k, _):
        cid = k * NSUB + s

        @pl.when(cid < NZC)
        def _():
            pltpu.sync_copy(acc.at[pl.ds(cid * ZCH, ZCH)],
                            out_hbm.at[pl.ds(cbase + cid * ZCH, ZCH)])
        return 0
    lax.fori_loop(0, (NZC + NSUB - 1) // NSUB, _flush_chunk, 0)


def _conv(xi2, w2, src, dst):
    mesh = plsc.VectorSubcoreMesh(core_axis_name="c", subcore_axis_name="s")
    kfn = functools.partial(
        pl.kernel, mesh=mesh,
        compiler_params=pltpu.CompilerParams(use_tc_tiling_on_sc=False),
        out_type=jax.ShapeDtypeStruct((2 * N, HDIM), jnp.float32),
        scratch_types=(
            [pltpu.VMEM((CH,), jnp.int32)] * 9
            + [pltpu.VMEM((CH, HDIM), jnp.float32)] * 6
            + [
                pltpu.VMEM((ZCH, HDIM), jnp.float32),
                pltpu.VMEM_SHARED((N, HDIM), jnp.float32),
            ]
            + [pltpu.SemaphoreType.DMA] * 9
        ),
    )(_conv_body)
    return kfn(xi2, w2, src, dst)


def _post_body_mid(agg_ref, xip_ref, w2_ref, b2_ref, w3_ref, b3_ref,
                   w1n_ref, b1n_ref, xi_ref, xs_ref):
    a = jnp.concatenate([agg_ref[0], agg_ref[1]], axis=1)
    t = _softplus(jnp.dot(a, w2_ref[...], preferred_element_type=jnp.float32) + b2_ref[...])
    y = jnp.dot(t, w3_ref[...], preferred_element_type=jnp.float32) + b3_ref[...] + xip_ref[...]
    xi_ref[...] = y
    z = jnp.dot(y, w1n_ref[...], preferred_element_type=jnp.float32) + b1n_ref[...]
    xs_ref[0] = z[:, :HDIM]
    xs_ref[1] = z[:, HDIM:]


def _post_body_last(agg_ref, xip_ref, w2_ref, b2_ref, w3_ref, b3_ref, xi_ref):
    a = jnp.concatenate([agg_ref[0], agg_ref[1]], axis=1)
    t = _softplus(jnp.dot(a, w2_ref[...], preferred_element_type=jnp.float32) + b2_ref[...])
    y = jnp.dot(t, w3_ref[...], preferred_element_type=jnp.float32) + b3_ref[...] + xip_ref[...]
    xi_ref[...] = y


_MAT_SPEC = pl.BlockSpec((DIM, DIM), lambda i: (0, 0))
_VEC_SPEC = pl.BlockSpec((1, DIM), lambda i: (0, 0))


def _post_mid(agg3, xi_prev, w2, b2r, w3, b3r, w1n, b1nr):
    return pl.pallas_call(
        _post_body_mid,
        grid=(N // RN,),
        in_specs=[
            pl.BlockSpec((2, RN, HDIM), lambda i: (0, i, 0)),
            pl.BlockSpec((RN, DIM), lambda i: (i, 0)),
            _MAT_SPEC, _VEC_SPEC, _MAT_SPEC, _VEC_SPEC, _MAT_SPEC, _VEC_SPEC,
        ],
        out_specs=[
            pl.BlockSpec((RN, DIM), lambda i: (i, 0)),
            pl.BlockSpec((2, RN, HDIM), lambda i: (0, i, 0)),
        ],
        out_shape=[
            jax.ShapeDtypeStruct((N, DIM), jnp.float32),
            jax.ShapeDtypeStruct((2, N, HDIM), jnp.float32),
        ],
    )(agg3, xi_prev, w2, b2r, w3, b3r, w1n, b1nr)


def _post_last(agg3, xi_prev, w2, b2r, w3, b3r):
    return pl.pallas_call(
        _post_body_last,
        grid=(N // RN,),
        in_specs=[
            pl.BlockSpec((2, RN, HDIM), lambda i: (0, i, 0)),
            pl.BlockSpec((RN, DIM), lambda i: (i, 0)),
            _MAT_SPEC, _VEC_SPEC, _MAT_SPEC, _VEC_SPEC,
        ],
        out_specs=pl.BlockSpec((RN, DIM), lambda i: (i, 0)),
        out_shape=jax.ShapeDtypeStruct((N, DIM), jnp.float32),
    )(agg3, xi_prev, w2, b2r, w3, b3r)


def kernel(species, edge_src, edge_dst, distances, species_table,
           W1, b1, Wf1, bf1, Wf2, bf2, W2, b2, W3, b3):
    species3 = species.astype(jnp.int32).reshape(N // RN, 1, RN)
    src = edge_src.astype(jnp.int32)
    dst = edge_dst.astype(jnp.int32)
    dist3 = distances.reshape(E // EBLK, 1, EBLK)
    tbl128 = jnp.zeros((128, DIM), jnp.float32).at[:100].set(species_table)

    b1r = b1.reshape(NL, 1, DIM)
    bf1r = bf1.reshape(NL, 1, DIM)
    bf2r = bf2.reshape(NL, 1, DIM)
    b2r = b2.reshape(NL, 1, DIM)
    b3r = b3.reshape(NL, 1, DIM)

    w_all = _filters(dist3, Wf1, bf1r, Wf2, bf2r)
    xi_prev, xs = _encode(species3, tbl128, W1[0], b1r[0])

    for l in range(NL):
        agg = _conv(xs.reshape(2 * N, HDIM), w_all[l], src, dst)
        agg3 = agg.reshape(2, N, HDIM)
        if l + 1 < NL:
            xi_prev, xs = _post_mid(agg3, xi_prev, W2[l], b2r[l], W3[l], b3r[l],
                                    W1[l + 1], b1r[l + 1])
        else:
            xi_prev = _post_last(agg3, xi_prev, W2[l], b2r[l], W3[l], b3r[l])
    return xi_prev


# speedup vs baseline: 3.9456x; 1.3644x over previous
import functools

import jax
import jax.numpy as jnp
import numpy as np
from jax import lax
from jax.experimental import pallas as pl
from jax.experimental.pallas import tpu as pltpu
from jax.experimental.pallas import tpu_sc as plsc

N = 50000
E = 800000
DIM = 64
HDIM = DIM // 2
NB = 16
NL = 3
CUTOFF = 5.0

RN = 2000
EBLK = 3200

NCORE = 2
NSUB = 16
CH = 80
NCHT = E // CH
MPT = NCHT // NSUB
NKIT = (MPT + 2) // 3
ZCH = 80
NZC = N // ZCH


def _softplus(x):
    return jnp.maximum(x, 0.0) + jnp.log1p(jnp.exp(-jnp.abs(x)))


def _encode_body(sp_ref, tbl_ref, w1_ref, b1_ref, xi_ref, xs_ref):
    sp = sp_ref[0, 0, :]
    oh = (sp[:, None] == lax.broadcasted_iota(jnp.int32, (1, 128), 1)).astype(jnp.float32)
    xi0 = jnp.dot(oh, tbl_ref[...], preferred_element_type=jnp.float32)
    xi_ref[...] = xi0
    y = jnp.dot(xi0, w1_ref[...], preferred_element_type=jnp.float32) + b1_ref[...]
    xs_ref[0] = y[:, :HDIM]
    xs_ref[1] = y[:, HDIM:]


def _encode(species3, tbl128, w1, b1r):
    return pl.pallas_call(
        _encode_body,
        grid=(N // RN,),
        in_specs=[
            pl.BlockSpec((1, 1, RN), lambda i: (i, 0, 0)),
            pl.BlockSpec((128, DIM), lambda i: (0, 0)),
            pl.BlockSpec((DIM, DIM), lambda i: (0, 0)),
            pl.BlockSpec((1, DIM), lambda i: (0, 0)),
        ],
        out_specs=[
            pl.BlockSpec((RN, DIM), lambda i: (i, 0)),
            pl.BlockSpec((2, RN, HDIM), lambda i: (0, i, 0)),
        ],
        out_shape=[
            jax.ShapeDtypeStruct((N, DIM), jnp.float32),
            jax.ShapeDtypeStruct((2, N, HDIM), jnp.float32),
        ],
    )(species3, tbl128, w1, b1r)


EBLK4 = EBLK // 4


def _filters_body(d_ref, wf1_ref, bf1_ref, wf2lo_ref, bf2lo_ref,
                  wf2hi_ref, bf2hi_ref, lo_ref, hi_ref):
    mu = lax.broadcasted_iota(jnp.int32, (1, NB), 1).astype(jnp.float32) * (
        CUTOFF / (NB - 1))
    sigma = CUTOFF / NB
    rbs = []
    for q in range(4):
        d = d_ref[0, q, :]
        rbs.append(jnp.exp(-((d[:, None] - mu) ** 2) * (1.0 / (2.0 * sigma * sigma))))
    rb_pack = jnp.concatenate(rbs, axis=1)
    h = _softplus(jnp.dot(rb_pack, wf1_ref[...], preferred_element_type=jnp.float32)
                  + bf1_ref[...])
    lo_ref[...] = jnp.dot(h, wf2lo_ref[...], preferred_element_type=jnp.float32) + bf2lo_ref[...]
    hi_ref[...] = jnp.dot(h, wf2hi_ref[...], preferred_element_type=jnp.float32) + bf2hi_ref[...]


def _filters(dT3, wf1b, bf1b, wf2lob, bf2lob, wf2hib, bf2hib):
    return pl.pallas_call(
        _filters_body,
        grid=(E // EBLK,),
        in_specs=[
            pl.BlockSpec((1, 4, EBLK4), lambda i: (i, 0, 0)),
            pl.BlockSpec((4 * NB, 4 * DIM), lambda i: (0, 0)),
            pl.BlockSpec((1, 4 * DIM), lambda i: (0, 0)),
            pl.BlockSpec((4 * DIM, 128), lambda i: (0, 0)),
            pl.BlockSpec((1, 128), lambda i: (0, 0)),
            pl.BlockSpec((4 * DIM, 128), lambda i: (0, 0)),
            pl.BlockSpec((1, 128), lambda i: (0, 0)),
        ],
        out_specs=[pl.BlockSpec((EBLK4, 128), lambda i: (i, 0))] * 2,
        out_shape=[jax.ShapeDtypeStruct((E // 4, 128), jnp.float32)] * 2,
    )(dT3, wf1b, bf1b, wf2lob, bf2lob, wf2hib, bf2hib)


def _conv_body(xi2_hbm, wlo_hbm, whi_hbm, src_hbm, dst_hbm, out_hbm,
               d0, d1, d2, sr0, sr1, sr2, ix0, ix1, ix2,
               w0, w1, w2, r0, r1, r2, zbuf, acc,
               f0, f1, f2, g0, g1, g2, s0, s1, s2):
    c = lax.axis_index("c")
    s = lax.axis_index("s")
    cbase = c * N
    dstv = (d0, d1, d2)
    srcv = (sr0, sr1, sr2)
    idxv = (ix0, ix1, ix2)
    wv = (w0, w1, w2)
    rowsv = (r0, r1, r2)
    fsem = (f0, f1, f2)
    gsem = (g0, g1, g2)
    ssem = (s0, s1, s2)

    def _zz(i, _):
        zbuf[i, pl.ds(0, 16)] = jnp.zeros((16,), jnp.float32)
        zbuf[i, pl.ds(16, 16)] = jnp.zeros((16,), jnp.float32)
        return 0
    lax.fori_loop(0, ZCH, _zz, 0)

    def _zero_chunk(k, _):
        cid = k * NSUB + s

        @pl.when(cid < NZC)
        def _():
            pltpu.sync_copy(zbuf, acc.at[pl.ds(cid * ZCH, ZCH)])
        return 0
    lax.fori_loop(0, (NZC + NSUB - 1) // NSUB, _zero_chunk, 0)
    plsc.subcore_barrier()

    def _e0(m):
        return (s + NSUB * m) * CH

    def _valid(m):
        return m < MPT

    def _fetch_copies(j, m, w_hbm):
        e0 = _e0(m)
        return (
            pltpu.make_async_copy(dst_hbm.at[pl.ds(e0, CH)], dstv[j], fsem[j]),
            pltpu.make_async_copy(src_hbm.at[pl.ds(e0, CH)], srcv[j], fsem[j]),
            pltpu.make_async_copy(w_hbm.at[pl.ds(e0 * HDIM, CH * HDIM)], wv[j], fsem[j]),
        )

    def _fetch_issue(j, m):
        @pl.when(c == 0)
        def _():
            for cp in _fetch_copies(j, m, wlo_hbm):
                cp.start()

        @pl.when(c == 1)
        def _():
            for cp in _fetch_copies(j, m, whi_hbm):
                cp.start()

    def _fetch_wait(j, m):
        @pl.when(c == 0)
        def _():
            for cp in _fetch_copies(j, m, wlo_hbm):
                cp.wait()

        @pl.when(c == 1)
        def _():
            for cp in _fetch_copies(j, m, whi_hbm):
                cp.wait()

    def _idx_compute(j):
        for q in range(CH // 16):
            idxv[j][pl.ds(q * 16, 16)] = dstv[j][pl.ds(q * 16, 16)] + cbase

    def _gather(j):
        return pltpu.make_async_copy(xi2_hbm.at[idxv[j]], rowsv[j], gsem[j])

    def _scatter(j):
        return pltpu.make_async_copy(rowsv[j], acc.at[srcv[j]], ssem[j])

    def _mul(j):
        @plsc.parallel_loop(0, CH, unroll=4)
        def _(r):
            rowsv[j][r, pl.ds(0, 16)] = rowsv[j][r, pl.ds(0, 16)] * wv[j][pl.ds(r * HDIM, 16)]
            rowsv[j][r, pl.ds(16, 16)] = rowsv[j][r, pl.ds(16, 16)] * wv[j][pl.ds(r * HDIM + 16, 16)]

    _fetch_issue(0, 0)
    _fetch_wait(0, 0)
    _idx_compute(0)
    _gather(0).start()
    _fetch_issue(1, 1)

    def _step(k, _):
        for jj in range(3):
            m = 3 * k + jj
            jn = (jj + 1) % 3
            jp = (jj + 2) % 3

            @pl.when(_valid(m + 1))
            def _():
                _fetch_wait(jn, m + 1)
                _idx_compute(jn)
                _gather(jn).start()

            @pl.when(_valid(m))
            def _():
                _gather(jj).wait()
                _mul(jj)
                _scatter(jj).start(add=True)

            @pl.when((m >= 1) & _valid(m - 1))
            def _():
                _scatter(jp).wait()

            @pl.when(_valid(m + 2))
            def _():
                _fetch_issue(jp, m + 2)
        return 0

    lax.fori_loop(0, NKIT, _step, 0)
    plsc.subcore_barrier()

    def _flush_chunk(k, _):
        cid = k * NSUB + s

        @pl.when(cid < NZC)
        def _():
            pltpu.sync_copy(acc.at[pl.ds(cid * ZCH, ZCH)],
                            out_hbm.at[pl.ds(cbase + cid * ZCH, ZCH)])
        return 0
    lax.fori_loop(0, (NZC + NSUB - 1) // NSUB, _flush_chunk, 0)


def _conv(xi2, wlo, whi, src, dst):
    mesh = plsc.VectorSubcoreMesh(core_axis_name="c", subcore_axis_name="s")
    kfn = functools.partial(
        pl.kernel, mesh=mesh,
        compiler_params=pltpu.CompilerParams(use_tc_tiling_on_sc=False),
        out_type=jax.ShapeDtypeStruct((2 * N, HDIM), jnp.float32),
        scratch_types=(
            [pltpu.VMEM((CH,), jnp.int32)] * 9
            + [pltpu.VMEM((CH * HDIM,), jnp.float32)] * 3
            + [pltpu.VMEM((CH, HDIM), jnp.float32)] * 3
            + [
                pltpu.VMEM((ZCH, HDIM), jnp.float32),
                pltpu.VMEM_SHARED((N, HDIM), jnp.float32),
            ]
            + [pltpu.SemaphoreType.DMA] * 9
        ),
    )(_conv_body)
    return kfn(xi2, wlo, whi, src, dst)


def _post_body_mid(agg_ref, xip_ref, w2_ref, b2_ref, w3_ref, b3_ref,
                   w1n_ref, b1n_ref, xi_ref, xs_ref):
    a = jnp.concatenate([agg_ref[0], agg_ref[1]], axis=1)
    t = _softplus(jnp.dot(a, w2_ref[...], preferred_element_type=jnp.float32) + b2_ref[...])
    y = jnp.dot(t, w3_ref[...], preferred_element_type=jnp.float32) + b3_ref[...] + xip_ref[...]
    xi_ref[...] = y
    z = jnp.dot(y, w1n_ref[...], preferred_element_type=jnp.float32) + b1n_ref[...]
    xs_ref[0] = z[:, :HDIM]
    xs_ref[1] = z[:, HDIM:]


def _post_body_last(agg_ref, xip_ref, w2_ref, b2_ref, w3_ref, b3_ref, xi_ref):
    a = jnp.concatenate([agg_ref[0], agg_ref[1]], axis=1)
    t = _softplus(jnp.dot(a, w2_ref[...], preferred_element_type=jnp.float32) + b2_ref[...])
    y = jnp.dot(t, w3_ref[...], preferred_element_type=jnp.float32) + b3_ref[...] + xip_ref[...]
    xi_ref[...] = y


_MAT_SPEC = pl.BlockSpec((DIM, DIM), lambda i: (0, 0))
_VEC_SPEC = pl.BlockSpec((1, DIM), lambda i: (0, 0))


def _post_mid(agg3, xi_prev, w2, b2r, w3, b3r, w1n, b1nr):
    return pl.pallas_call(
        _post_body_mid,
        grid=(N // RN,),
        in_specs=[
            pl.BlockSpec((2, RN, HDIM), lambda i: (0, i, 0)),
            pl.BlockSpec((RN, DIM), lambda i: (i, 0)),
            _MAT_SPEC, _VEC_SPEC, _MAT_SPEC, _VEC_SPEC, _MAT_SPEC, _VEC_SPEC,
        ],
        out_specs=[
            pl.BlockSpec((RN, DIM), lambda i: (i, 0)),
            pl.BlockSpec((2, RN, HDIM), lambda i: (0, i, 0)),
        ],
        out_shape=[
            jax.ShapeDtypeStruct((N, DIM), jnp.float32),
            jax.ShapeDtypeStruct((2, N, HDIM), jnp.float32),
        ],
    )(agg3, xi_prev, w2, b2r, w3, b3r, w1n, b1nr)


def _post_last(agg3, xi_prev, w2, b2r, w3, b3r):
    return pl.pallas_call(
        _post_body_last,
        grid=(N // RN,),
        in_specs=[
            pl.BlockSpec((2, RN, HDIM), lambda i: (0, i, 0)),
            pl.BlockSpec((RN, DIM), lambda i: (i, 0)),
            _MAT_SPEC, _VEC_SPEC, _MAT_SPEC, _VEC_SPEC,
        ],
        out_specs=pl.BlockSpec((RN, DIM), lambda i: (i, 0)),
        out_shape=jax.ShapeDtypeStruct((N, DIM), jnp.float32),
    )(agg3, xi_prev, w2, b2r, w3, b3r)


def kernel(species, edge_src, edge_dst, distances, species_table,
           W1, b1, Wf1, bf1, Wf2, bf2, W2, b2, W3, b3):
    species3 = species.astype(jnp.int32).reshape(N // RN, 1, RN)
    src = edge_src.astype(jnp.int32)
    dst = edge_dst.astype(jnp.int32)
    dT3 = distances.reshape(E // 4, 4).T.reshape(4, E // EBLK, EBLK4).transpose(1, 0, 2)
    tbl128 = jnp.zeros((128, DIM), jnp.float32).at[:100].set(species_table)

    b1r = b1.reshape(NL, 1, DIM)
    bf1r = bf1.reshape(NL, 1, DIM)
    bf2r = bf2.reshape(NL, 1, DIM)
    b2r = b2.reshape(NL, 1, DIM)
    b3r = b3.reshape(NL, 1, DIM)

    m1 = jnp.asarray(np.kron(np.eye(4, dtype=np.float32), np.ones((NB, DIM), np.float32)))
    w_all = []
    for l in range(NL):
        wf1b = jnp.tile(Wf1[l], (4, 4)) * m1
        bf1b = jnp.tile(bf1[l], 4).reshape(1, 4 * DIM)
        m2 = jnp.asarray(np.kron(np.eye(4, dtype=np.float32), np.ones((DIM, HDIM), np.float32)))
        wf2lob = jnp.tile(Wf2[l][:, :HDIM], (4, 4)) * m2
        wf2hib = jnp.tile(Wf2[l][:, HDIM:], (4, 4)) * m2
        bf2lob = jnp.tile(bf2[l][:HDIM], 4).reshape(1, 128)
        bf2hib = jnp.tile(bf2[l][HDIM:], 4).reshape(1, 128)
        wlo, whi = _filters(dT3, wf1b, bf1b, wf2lob, bf2lob, wf2hib, bf2hib)
        w_all.append((wlo.reshape(E * HDIM), whi.reshape(E * HDIM)))

    xi_prev, xs = _encode(species3, tbl128, W1[0], b1r[0])

    for l in range(NL):
        agg = _conv(xs.reshape(2 * N, HDIM), w_all[l][0], w_all[l][1], src, dst)
        agg3 = agg.reshape(2, N, HDIM)
        if l + 1 < NL:
            xi_prev, xs = _post_mid(agg3, xi_prev, W2[l], b2r[l], W3[l], b3r[l],
                                    W1[l + 1], b1r[l + 1])
        else:
            xi_prev = _post_last(agg3, xi_prev, W2[l], b2r[l], W3[l], b3r[l])
    return xi_prev


# speedup vs baseline: 4.6752x; 1.1849x over previous
import functools

import jax
import jax.numpy as jnp
import numpy as np
from jax import lax
from jax.experimental import pallas as pl
from jax.experimental.pallas import tpu as pltpu
from jax.experimental.pallas import tpu_sc as plsc

N = 50000
E = 800000
DIM = 64
HDIM = DIM // 2
NB = 16
NL = 3
CUTOFF = 5.0

RN = 2000
EBLK = 3200

NCORE = 2
NSUB = 16
CH = 128
NCHT = E // CH
MAXM = (NCHT + NSUB - 1) // NSUB
NKIT = (MAXM + 2) // 3
ZCH = 80
NZC = N // ZCH


def _softplus(x):
    return jnp.maximum(x, 0.0) + jnp.log1p(jnp.exp(-jnp.abs(x)))


def _encode_body(sp_ref, tbl_ref, w1_ref, b1_ref, xi_ref, xs_ref):
    sp = sp_ref[0, 0, :]
    oh = (sp[:, None] == lax.broadcasted_iota(jnp.int32, (1, 128), 1)).astype(jnp.float32)
    xi0 = jnp.dot(oh, tbl_ref[...], preferred_element_type=jnp.float32)
    xi_ref[...] = xi0
    y = jnp.dot(xi0, w1_ref[...], preferred_element_type=jnp.float32) + b1_ref[...]
    xs_ref[0] = y[:, :HDIM]
    xs_ref[1] = y[:, HDIM:]


def _encode(species3, tbl128, w1, b1r):
    return pl.pallas_call(
        _encode_body,
        grid=(N // RN,),
        in_specs=[
            pl.BlockSpec((1, 1, RN), lambda i: (i, 0, 0)),
            pl.BlockSpec((128, DIM), lambda i: (0, 0)),
            pl.BlockSpec((DIM, DIM), lambda i: (0, 0)),
            pl.BlockSpec((1, DIM), lambda i: (0, 0)),
        ],
        out_specs=[
            pl.BlockSpec((RN, DIM), lambda i: (i, 0)),
            pl.BlockSpec((2, RN, HDIM), lambda i: (0, i, 0)),
        ],
        out_shape=[
            jax.ShapeDtypeStruct((N, DIM), jnp.float32),
            jax.ShapeDtypeStruct((2, N, HDIM), jnp.float32),
        ],
    )(species3, tbl128, w1, b1r)


EBLK4 = EBLK // 4


def _filters_body(d_ref, wf1_ref, bf1_ref, wf2lo_ref, bf2lo_ref,
                  wf2hi_ref, bf2hi_ref, lo_ref, hi_ref):
    mu = lax.broadcasted_iota(jnp.int32, (1, NB), 1).astype(jnp.float32) * (
        CUTOFF / (NB - 1))
    sigma = CUTOFF / NB
    rbs = []
    for q in range(4):
        d = d_ref[0, q, :]
        rbs.append(jnp.exp(-((d[:, None] - mu) ** 2) * (1.0 / (2.0 * sigma * sigma))))
    rb_pack = jnp.concatenate(rbs, axis=1)
    h = _softplus(jnp.dot(rb_pack, wf1_ref[...], preferred_element_type=jnp.float32)
                  + bf1_ref[...])
    lo_ref[...] = jnp.dot(h, wf2lo_ref[...], preferred_element_type=jnp.float32) + bf2lo_ref[...]
    hi_ref[...] = jnp.dot(h, wf2hi_ref[...], preferred_element_type=jnp.float32) + bf2hi_ref[...]


def _filters(dT3, wf1b, bf1b, wf2lob, bf2lob, wf2hib, bf2hib):
    return pl.pallas_call(
        _filters_body,
        grid=(E // EBLK,),
        in_specs=[
            pl.BlockSpec((1, 4, EBLK4), lambda i: (i, 0, 0)),
            pl.BlockSpec((4 * NB, 4 * DIM), lambda i: (0, 0)),
            pl.BlockSpec((1, 4 * DIM), lambda i: (0, 0)),
            pl.BlockSpec((4 * DIM, 128), lambda i: (0, 0)),
            pl.BlockSpec((1, 128), lambda i: (0, 0)),
            pl.BlockSpec((4 * DIM, 128), lambda i: (0, 0)),
            pl.BlockSpec((1, 128), lambda i: (0, 0)),
        ],
        out_specs=[pl.BlockSpec((EBLK4, 128), lambda i: (i, 0))] * 2,
        out_shape=[jax.ShapeDtypeStruct((E // 4, 128), jnp.float32)] * 2,
    )(dT3, wf1b, bf1b, wf2lob, bf2lob, wf2hib, bf2hib)


def _conv_body(xi2_hbm, wlo_hbm, whi_hbm, src_hbm, dst_hbm, out_hbm,
               d0, d1, d2, sr0, sr1, sr2, ix0, ix1, ix2,
               w0, w1, w2, r0, r1, r2, zbuf, acc,
               f0, f1, f2, g0, g1, g2, s0, s1, s2):
    c = lax.axis_index("c")
    s = lax.axis_index("s")
    cbase = c * N
    dstv = (d0, d1, d2)
    srcv = (sr0, sr1, sr2)
    idxv = (ix0, ix1, ix2)
    wv = (w0, w1, w2)
    rowsv = (r0, r1, r2)
    fsem = (f0, f1, f2)
    gsem = (g0, g1, g2)
    ssem = (s0, s1, s2)

    def _zz(i, _):
        zbuf[i, pl.ds(0, 16)] = jnp.zeros((16,), jnp.float32)
        zbuf[i, pl.ds(16, 16)] = jnp.zeros((16,), jnp.float32)
        return 0
    lax.fori_loop(0, ZCH, _zz, 0)

    def _zero_chunk(k, _):
        cid = k * NSUB + s

        @pl.when(cid < NZC)
        def _():
            pltpu.sync_copy(zbuf, acc.at[pl.ds(cid * ZCH, ZCH)])
        return 0
    lax.fori_loop(0, (NZC + NSUB - 1) // NSUB, _zero_chunk, 0)
    plsc.subcore_barrier()

    def _e0(m):
        return (s + NSUB * m) * CH

    def _valid(m):
        return (s + NSUB * m) < NCHT

    def _fetch_copies(j, m, w_hbm):
        e0 = _e0(m)
        return (
            pltpu.make_async_copy(dst_hbm.at[pl.ds(e0, CH)], dstv[j], fsem[j]),
            pltpu.make_async_copy(src_hbm.at[pl.ds(e0, CH)], srcv[j], fsem[j]),
            pltpu.make_async_copy(w_hbm.at[pl.ds(e0 * HDIM, CH * HDIM)], wv[j], fsem[j]),
        )

    def _fetch_issue(j, m):
        @pl.when(c == 0)
        def _():
            for cp in _fetch_copies(j, m, wlo_hbm):
                cp.start()

        @pl.when(c == 1)
        def _():
            for cp in _fetch_copies(j, m, whi_hbm):
                cp.start()

    def _fetch_wait(j, m):
        @pl.when(c == 0)
        def _():
            for cp in _fetch_copies(j, m, wlo_hbm):
                cp.wait()

        @pl.when(c == 1)
        def _():
            for cp in _fetch_copies(j, m, whi_hbm):
                cp.wait()

    def _idx_compute(j):
        for q in range(CH // 16):
            idxv[j][pl.ds(q * 16, 16)] = dstv[j][pl.ds(q * 16, 16)] + cbase

    def _gather(j):
        return pltpu.make_async_copy(xi2_hbm.at[idxv[j]], rowsv[j], gsem[j])

    def _scatter(j):
        return pltpu.make_async_copy(rowsv[j], acc.at[srcv[j]], ssem[j])

    def _mul(j):
        @plsc.parallel_loop(0, CH, unroll=8)
        def _(r):
            rowsv[j][r, pl.ds(0, 16)] = rowsv[j][r, pl.ds(0, 16)] * wv[j][pl.ds(r * HDIM, 16)]
            rowsv[j][r, pl.ds(16, 16)] = rowsv[j][r, pl.ds(16, 16)] * wv[j][pl.ds(r * HDIM + 16, 16)]

    _fetch_issue(0, 0)
    _fetch_wait(0, 0)
    _idx_compute(0)
    _gather(0).start()
    _fetch_issue(1, 1)

    def _step(k, _):
        for jj in range(3):
            m = 3 * k + jj
            jn = (jj + 1) % 3
            jp = (jj + 2) % 3

            @pl.when(_valid(m + 1))
            def _():
                _fetch_wait(jn, m + 1)
                _idx_compute(jn)
                _gather(jn).start()

            @pl.when(_valid(m))
            def _():
                _gather(jj).wait()
                _mul(jj)
                _scatter(jj).start(add=True)

            @pl.when((m >= 1) & _valid(m - 1))
            def _():
                _scatter(jp).wait()

            @pl.when(_valid(m + 2))
            def _():
                _fetch_issue(jp, m + 2)
        return 0

    lax.fori_loop(0, NKIT, _step, 0)
    plsc.subcore_barrier()

    def _flush_chunk(k, _):
        cid = k * NSUB + s

        @pl.when(cid < NZC)
        def _():
            pltpu.sync_copy(acc.at[pl.ds(cid * ZCH, ZCH)],
                            out_hbm.at[pl.ds(cbase + cid * ZCH, ZCH)])
        return 0
    lax.fori_loop(0, (NZC + NSUB - 1) // NSUB, _flush_chunk, 0)


def _conv(xi2, wlo, whi, src, dst):
    mesh = plsc.VectorSubcoreMesh(core_axis_name="c", subcore_axis_name="s")
    kfn = functools.partial(
        pl.kernel, mesh=mesh,
        compiler_params=pltpu.CompilerParams(use_tc_tiling_on_sc=False),
        out_type=jax.ShapeDtypeStruct((2 * N, HDIM), jnp.float32),
        scratch_types=(
            [pltpu.VMEM((CH,), jnp.int32)] * 9
            + [pltpu.VMEM((CH * HDIM,), jnp.float32)] * 3
            + [pltpu.VMEM((CH, HDIM), jnp.float32)] * 3
            + [
                pltpu.VMEM((ZCH, HDIM), jnp.float32),
                pltpu.VMEM_SHARED((N, HDIM), jnp.float32),
            ]
            + [pltpu.SemaphoreType.DMA] * 9
        ),
    )(_conv_body)
    return kfn(xi2, wlo, whi, src, dst)


def _post_body_mid(agg_ref, xip_ref, w2_ref, b2_ref, w3_ref, b3_ref,
                   w1n_ref, b1n_ref, xi_ref, xs_ref):
    a = jnp.concatenate([agg_ref[0], agg_ref[1]], axis=1)
    t = _softplus(jnp.dot(a, w2_ref[...], preferred_element_type=jnp.float32) + b2_ref[...])
    y = jnp.dot(t, w3_ref[...], preferred_element_type=jnp.float32) + b3_ref[...] + xip_ref[...]
    xi_ref[...] = y
    z = jnp.dot(y, w1n_ref[...], preferred_element_type=jnp.float32) + b1n_ref[...]
    xs_ref[0] = z[:, :HDIM]
    xs_ref[1] = z[:, HDIM:]


def _post_body_last(agg_ref, xip_ref, w2_ref, b2_ref, w3_ref, b3_ref, xi_ref):
    a = jnp.concatenate([agg_ref[0], agg_ref[1]], axis=1)
    t = _softplus(jnp.dot(a, w2_ref[...], preferred_element_type=jnp.float32) + b2_ref[...])
    y = jnp.dot(t, w3_ref[...], preferred_element_type=jnp.float32) + b3_ref[...] + xip_ref[...]
    xi_ref[...] = y


_MAT_SPEC = pl.BlockSpec((DIM, DIM), lambda i: (0, 0))
_VEC_SPEC = pl.BlockSpec((1, DIM), lambda i: (0, 0))


def _post_mid(agg3, xi_prev, w2, b2r, w3, b3r, w1n, b1nr):
    return pl.pallas_call(
        _post_body_mid,
        grid=(N // RN,),
        in_specs=[
            pl.BlockSpec((2, RN, HDIM), lambda i: (0, i, 0)),
            pl.BlockSpec((RN, DIM), lambda i: (i, 0)),
            _MAT_SPEC, _VEC_SPEC, _MAT_SPEC, _VEC_SPEC, _MAT_SPEC, _VEC_SPEC,
        ],
        out_specs=[
            pl.BlockSpec((RN, DIM), lambda i: (i, 0)),
            pl.BlockSpec((2, RN, HDIM), lambda i: (0, i, 0)),
        ],
        out_shape=[
            jax.ShapeDtypeStruct((N, DIM), jnp.float32),
            jax.ShapeDtypeStruct((2, N, HDIM), jnp.float32),
        ],
    )(agg3, xi_prev, w2, b2r, w3, b3r, w1n, b1nr)


def _post_last(agg3, xi_prev, w2, b2r, w3, b3r):
    return pl.pallas_call(
        _post_body_last,
        grid=(N // RN,),
        in_specs=[
            pl.BlockSpec((2, RN, HDIM), lambda i: (0, i, 0)),
            pl.BlockSpec((RN, DIM), lambda i: (i, 0)),
            _MAT_SPEC, _VEC_SPEC, _MAT_SPEC, _VEC_SPEC,
        ],
        out_specs=pl.BlockSpec((RN, DIM), lambda i: (i, 0)),
        out_shape=jax.ShapeDtypeStruct((N, DIM), jnp.float32),
    )(agg3, xi_prev, w2, b2r, w3, b3r)


def kernel(species, edge_src, edge_dst, distances, species_table,
           W1, b1, Wf1, bf1, Wf2, bf2, W2, b2, W3, b3):
    species3 = species.astype(jnp.int32).reshape(N // RN, 1, RN)
    src = edge_src.astype(jnp.int32)
    dst = edge_dst.astype(jnp.int32)
    dist3 = distances.reshape(E // 4, 4).T.reshape(4, E // EBLK, EBLK4).transpose(1, 0, 2)
    tbl128 = jnp.zeros((128, DIM), jnp.float32).at[:100].set(species_table)

    b1r = b1.reshape(NL, 1, DIM)
    bf1r = bf1.reshape(NL, 1, DIM)
    bf2r = bf2.reshape(NL, 1, DIM)
    b2r = b2.reshape(NL, 1, DIM)
    b3r = b3.reshape(NL, 1, DIM)

    m1 = jnp.asarray(np.kron(np.eye(4, dtype=np.float32), np.ones((NB, DIM), np.float32)))
    w_all = []
    for l in range(NL):
        wf1b = jnp.tile(Wf1[l], (4, 4)) * m1
        bf1b = jnp.tile(bf1[l], 4).reshape(1, 4 * DIM)
        m2 = jnp.asarray(np.kron(np.eye(4, dtype=np.float32), np.ones((DIM, HDIM), np.float32)))
        wf2lob = jnp.tile(Wf2[l][:, :HDIM], (4, 4)) * m2
        wf2hib = jnp.tile(Wf2[l][:, HDIM:], (4, 4)) * m2
        bf2lob = jnp.tile(bf2[l][:HDIM], 4).reshape(1, 128)
        bf2hib = jnp.tile(bf2[l][HDIM:], 4).reshape(1, 128)
        wlo, whi = _filters(dist3, wf1b, bf1b, wf2lob, bf2lob, wf2hib, bf2hib)
        w_all.append((wlo.reshape(E * HDIM), whi.reshape(E * HDIM)))

    xi_prev, xs = _encode(species3, tbl128, W1[0], b1r[0])

    for l in range(NL):
        agg = _conv(xs.reshape(2 * N, HDIM), w_all[l][0], w_all[l][1], src, dst)
        agg3 = agg.reshape(2, N, HDIM)
        if l + 1 < NL:
            xi_prev, xs = _post_mid(agg3, xi_prev, W2[l], b2r[l], W3[l], b3r[l],
                                    W1[l + 1], b1r[l + 1])
        else:
            xi_prev = _post_last(agg3, xi_prev, W2[l], b2r[l], W3[l], b3r[l])
    return xi_prev


# speedup vs baseline: 4.8855x; 1.0450x over previous
import functools

import jax
import jax.numpy as jnp
import numpy as np
from jax import lax
from jax.experimental import pallas as pl
from jax.experimental.pallas import tpu as pltpu
from jax.experimental.pallas import tpu_sc as plsc

N = 50000
E = 800000
DIM = 64
HDIM = DIM // 2
NB = 16
NL = 3
CUTOFF = 5.0

RN = 2000
EBLK = 3200

NCORE = 2
NSUB = 16
CH = 128
NCHT = E // CH
MAXM = (NCHT + NSUB - 1) // NSUB
NKIT = (MAXM + 2) // 3
ZCH = 80
NZC = N // ZCH


def _softplus(x):
    return jnp.maximum(x, 0.0) + jnp.log1p(jnp.exp(-jnp.abs(x)))


def _encode_body(sp_ref, tbl_ref, w1_ref, b1_ref, xi_ref, xs_ref):
    sp = sp_ref[0, 0, :]
    oh = (sp[:, None] == lax.broadcasted_iota(jnp.int32, (1, 128), 1)).astype(jnp.float32)
    xi0 = jnp.dot(oh, tbl_ref[...], preferred_element_type=jnp.float32)
    xi_ref[...] = xi0
    y = jnp.dot(xi0, w1_ref[...], preferred_element_type=jnp.float32) + b1_ref[...]
    xs_ref[0] = y[:, :HDIM]
    xs_ref[1] = y[:, HDIM:]


def _encode(species3, tbl128, w1, b1r):
    return pl.pallas_call(
        _encode_body,
        grid=(N // RN,),
        in_specs=[
            pl.BlockSpec((1, 1, RN), lambda i: (i, 0, 0)),
            pl.BlockSpec((128, DIM), lambda i: (0, 0)),
            pl.BlockSpec((DIM, DIM), lambda i: (0, 0)),
            pl.BlockSpec((1, DIM), lambda i: (0, 0)),
        ],
        out_specs=[
            pl.BlockSpec((RN, DIM), lambda i: (i, 0)),
            pl.BlockSpec((2, RN, HDIM), lambda i: (0, i, 0)),
        ],
        out_shape=[
            jax.ShapeDtypeStruct((N, DIM), jnp.float32),
            jax.ShapeDtypeStruct((2, N, HDIM), jnp.float32),
        ],
    )(species3, tbl128, w1, b1r)


EBLK4 = EBLK // 4


def _filters_body(d_ref, wf1_ref, bf1_ref, wf2lo_ref, bf2lo_ref,
                  wf2hi_ref, bf2hi_ref, lo_ref, hi_ref):
    mu = lax.broadcasted_iota(jnp.int32, (1, NB), 1).astype(jnp.float32) * (
        CUTOFF / (NB - 1))
    sigma = CUTOFF / NB
    dall = d_ref[0, 0, :]
    rbs = []
    for q in range(4):
        d = lax.slice(dall, (EBLK4 * q,), (EBLK4 * (q + 1),))
        rbs.append(jnp.exp(-((d[:, None] - mu) ** 2) * (1.0 / (2.0 * sigma * sigma))))
    rb_pack = jnp.concatenate(rbs, axis=1)
    h = _softplus(jnp.dot(rb_pack, wf1_ref[...], preferred_element_type=jnp.float32)
                  + bf1_ref[...])
    lo_ref[...] = jnp.dot(h, wf2lo_ref[...], preferred_element_type=jnp.float32) + bf2lo_ref[...]
    hi_ref[...] = jnp.dot(h, wf2hi_ref[...], preferred_element_type=jnp.float32) + bf2hi_ref[...]


def _filters(dT3, wf1b, bf1b, wf2lob, bf2lob, wf2hib, bf2hib):
    return pl.pallas_call(
        _filters_body,
        grid=(E // EBLK,),
        in_specs=[
            pl.BlockSpec((1, 1, EBLK), lambda i: (i, 0, 0)),
            pl.BlockSpec((4 * NB, 4 * DIM), lambda i: (0, 0)),
            pl.BlockSpec((1, 4 * DIM), lambda i: (0, 0)),
            pl.BlockSpec((4 * DIM, 128), lambda i: (0, 0)),
            pl.BlockSpec((1, 128), lambda i: (0, 0)),
            pl.BlockSpec((4 * DIM, 128), lambda i: (0, 0)),
            pl.BlockSpec((1, 128), lambda i: (0, 0)),
        ],
        out_specs=[pl.BlockSpec((EBLK4, 128), lambda i: (i, 0))] * 2,
        out_shape=[jax.ShapeDtypeStruct((E // 4, 128), jnp.float32)] * 2,
    )(dT3, wf1b, bf1b, wf2lob, bf2lob, wf2hib, bf2hib)


def _conv_body(xi2_hbm, wlo_hbm, whi_hbm, src_hbm, dst_hbm, out_hbm,
               d0, d1, d2, sr0, sr1, sr2, ix0, ix1, ix2,
               w0, w1, w2, r0, r1, r2, zbuf, acc,
               f0, f1, f2, g0, g1, g2, s0, s1, s2):
    c = lax.axis_index("c")
    s = lax.axis_index("s")
    cbase = c * N
    dstv = (d0, d1, d2)
    srcv = (sr0, sr1, sr2)
    idxv = (ix0, ix1, ix2)
    wv = (w0, w1, w2)
    rowsv = (r0, r1, r2)
    fsem = (f0, f1, f2)
    gsem = (g0, g1, g2)
    ssem = (s0, s1, s2)

    def _zz(i, _):
        zbuf[i, pl.ds(0, 16)] = jnp.zeros((16,), jnp.float32)
        zbuf[i, pl.ds(16, 16)] = jnp.zeros((16,), jnp.float32)
        return 0
    lax.fori_loop(0, ZCH, _zz, 0)

    def _zero_chunk(k, _):
        cid = k * NSUB + s

        @pl.when(cid < NZC)
        def _():
            pltpu.sync_copy(zbuf, acc.at[pl.ds(cid * ZCH, ZCH)])
        return 0
    lax.fori_loop(0, (NZC + NSUB - 1) // NSUB, _zero_chunk, 0)
    plsc.subcore_barrier()

    def _valid(m):
        return (s + NSUB * m) < NCHT

    QR = CH // 4

    def _fetch_copies(j, m, w_hbm):
        cid = s + NSUB * m
        i = cid // (EBLK4 // QR)
        t0 = (cid % (EBLK4 // QR)) * QR
        cps = [pltpu.make_async_copy(
            w_hbm.at[pl.ds(cid * (CH * HDIM), CH * HDIM)], wv[j], fsem[j])]
        for q in range(4):
            cps.append(pltpu.make_async_copy(
                dst_hbm.at[i, q, pl.ds(t0, QR)],
                dstv[j].at[pl.ds(q * QR, QR)], fsem[j]))
            cps.append(pltpu.make_async_copy(
                src_hbm.at[i, q, pl.ds(t0, QR)],
                srcv[j].at[pl.ds(q * QR, QR)], fsem[j]))
        return cps

    def _fetch_issue(j, m):
        @pl.when(c == 0)
        def _():
            for cp in _fetch_copies(j, m, wlo_hbm):
                cp.start()

        @pl.when(c == 1)
        def _():
            for cp in _fetch_copies(j, m, whi_hbm):
                cp.start()

    def _fetch_wait(j, m):
        @pl.when(c == 0)
        def _():
            for cp in _fetch_copies(j, m, wlo_hbm):
                cp.wait()

        @pl.when(c == 1)
        def _():
            for cp in _fetch_copies(j, m, whi_hbm):
                cp.wait()

    def _idx_compute(j):
        for q in range(CH // 16):
            idxv[j][pl.ds(q * 16, 16)] = dstv[j][pl.ds(q * 16, 16)] + cbase

    def _gather(j):
        return pltpu.make_async_copy(xi2_hbm.at[idxv[j]], rowsv[j], gsem[j])

    def _scatter(j):
        return pltpu.make_async_copy(rowsv[j], acc.at[srcv[j]], ssem[j])

    def _mul(j):
        @plsc.parallel_loop(0, QR, unroll=4)
        def _(t):
            for q in range(4):
                r = q * QR + t
                base = t * 128 + q * HDIM
                rowsv[j][r, pl.ds(0, 16)] = rowsv[j][r, pl.ds(0, 16)] * wv[j][pl.ds(base, 16)]
                rowsv[j][r, pl.ds(16, 16)] = rowsv[j][r, pl.ds(16, 16)] * wv[j][pl.ds(base + 16, 16)]

    _fetch_issue(0, 0)
    _fetch_wait(0, 0)
    _idx_compute(0)
    _gather(0).start()
    _fetch_issue(1, 1)

    def _step(k, _):
        for jj in range(3):
            m = 3 * k + jj
            jn = (jj + 1) % 3
            jp = (jj + 2) % 3

            @pl.when(_valid(m + 1))
            def _():
                _fetch_wait(jn, m + 1)
                _idx_compute(jn)
                _gather(jn).start()

            @pl.when(_valid(m))
            def _():
                _gather(jj).wait()
                _mul(jj)
                _scatter(jj).start(add=True)

            @pl.when((m >= 1) & _valid(m - 1))
            def _():
                _scatter(jp).wait()

            @pl.when(_valid(m + 2))
            def _():
                _fetch_issue(jp, m + 2)
        return 0

    lax.fori_loop(0, NKIT, _step, 0)
    plsc.subcore_barrier()

    def _flush_chunk(k, _):
        cid = k * NSUB + s

        @pl.when(cid < NZC)
        def _():
            pltpu.sync_copy(acc.at[pl.ds(cid * ZCH, ZCH)],
                            out_hbm.at[pl.ds(cbase + cid * ZCH, ZCH)])
        return 0
    lax.fori_loop(0, (NZC + NSUB - 1) // NSUB, _flush_chunk, 0)


def _conv(xi2, wlo, whi, src, dst):
    mesh = plsc.VectorSubcoreMesh(core_axis_name="c", subcore_axis_name="s")
    kfn = functools.partial(
        pl.kernel, mesh=mesh,
        compiler_params=pltpu.CompilerParams(use_tc_tiling_on_sc=False),
        out_type=jax.ShapeDtypeStruct((2 * N, HDIM), jnp.float32),
        scratch_types=(
            [pltpu.VMEM((CH,), jnp.int32)] * 9
            + [pltpu.VMEM((CH * HDIM,), jnp.float32)] * 3
            + [pltpu.VMEM((CH, HDIM), jnp.float32)] * 3
            + [
                pltpu.VMEM((ZCH, HDIM), jnp.float32),
                pltpu.VMEM_SHARED((N, HDIM), jnp.float32),
            ]
            + [pltpu.SemaphoreType.DMA] * 9
        ),
    )(_conv_body)
    return kfn(xi2, wlo, whi, src, dst)


def _post_body_mid(agg_ref, xip_ref, w2_ref, b2_ref, w3_ref, b3_ref,
                   w1n_ref, b1n_ref, xi_ref, xs_ref):
    a = jnp.concatenate([agg_ref[0], agg_ref[1]], axis=1)
    t = _softplus(jnp.dot(a, w2_ref[...], preferred_element_type=jnp.float32) + b2_ref[...])
    y = jnp.dot(t, w3_ref[...], preferred_element_type=jnp.float32) + b3_ref[...] + xip_ref[...]
    xi_ref[...] = y
    z = jnp.dot(y, w1n_ref[...], preferred_element_type=jnp.float32) + b1n_ref[...]
    xs_ref[0] = z[:, :HDIM]
    xs_ref[1] = z[:, HDIM:]


def _post_body_last(agg_ref, xip_ref, w2_ref, b2_ref, w3_ref, b3_ref, xi_ref):
    a = jnp.concatenate([agg_ref[0], agg_ref[1]], axis=1)
    t = _softplus(jnp.dot(a, w2_ref[...], preferred_element_type=jnp.float32) + b2_ref[...])
    y = jnp.dot(t, w3_ref[...], preferred_element_type=jnp.float32) + b3_ref[...] + xip_ref[...]
    xi_ref[...] = y


_MAT_SPEC = pl.BlockSpec((DIM, DIM), lambda i: (0, 0))
_VEC_SPEC = pl.BlockSpec((1, DIM), lambda i: (0, 0))


def _post_mid(agg3, xi_prev, w2, b2r, w3, b3r, w1n, b1nr):
    return pl.pallas_call(
        _post_body_mid,
        grid=(N // RN,),
        in_specs=[
            pl.BlockSpec((2, RN, HDIM), lambda i: (0, i, 0)),
            pl.BlockSpec((RN, DIM), lambda i: (i, 0)),
            _MAT_SPEC, _VEC_SPEC, _MAT_SPEC, _VEC_SPEC, _MAT_SPEC, _VEC_SPEC,
        ],
        out_specs=[
            pl.BlockSpec((RN, DIM), lambda i: (i, 0)),
            pl.BlockSpec((2, RN, HDIM), lambda i: (0, i, 0)),
        ],
        out_shape=[
            jax.ShapeDtypeStruct((N, DIM), jnp.float32),
            jax.ShapeDtypeStruct((2, N, HDIM), jnp.float32),
        ],
    )(agg3, xi_prev, w2, b2r, w3, b3r, w1n, b1nr)


def _post_last(agg3, xi_prev, w2, b2r, w3, b3r):
    return pl.pallas_call(
        _post_body_last,
        grid=(N // RN,),
        in_specs=[
            pl.BlockSpec((2, RN, HDIM), lambda i: (0, i, 0)),
            pl.BlockSpec((RN, DIM), lambda i: (i, 0)),
            _MAT_SPEC, _VEC_SPEC, _MAT_SPEC, _VEC_SPEC,
        ],
        out_specs=pl.BlockSpec((RN, DIM), lambda i: (i, 0)),
        out_shape=jax.ShapeDtypeStruct((N, DIM), jnp.float32),
    )(agg3, xi_prev, w2, b2r, w3, b3r)


def kernel(species, edge_src, edge_dst, distances, species_table,
           W1, b1, Wf1, bf1, Wf2, bf2, W2, b2, W3, b3):
    species3 = species.astype(jnp.int32).reshape(N // RN, 1, RN)
    src = edge_src.astype(jnp.int32)
    dst = edge_dst.astype(jnp.int32)
    dist3 = distances.reshape(E // EBLK, 1, EBLK)
    tbl128 = jnp.zeros((128, DIM), jnp.float32).at[:100].set(species_table)

    b1r = b1.reshape(NL, 1, DIM)
    bf1r = bf1.reshape(NL, 1, DIM)
    bf2r = bf2.reshape(NL, 1, DIM)
    b2r = b2.reshape(NL, 1, DIM)
    b3r = b3.reshape(NL, 1, DIM)

    m1 = jnp.asarray(np.kron(np.eye(4, dtype=np.float32), np.ones((NB, DIM), np.float32)))
    w_all = []
    for l in range(NL):
        wf1b = jnp.tile(Wf1[l], (4, 4)) * m1
        bf1b = jnp.tile(bf1[l], 4).reshape(1, 4 * DIM)
        m2 = jnp.asarray(np.kron(np.eye(4, dtype=np.float32), np.ones((DIM, HDIM), np.float32)))
        wf2lob = jnp.tile(Wf2[l][:, :HDIM], (4, 4)) * m2
        wf2hib = jnp.tile(Wf2[l][:, HDIM:], (4, 4)) * m2
        bf2lob = jnp.tile(bf2[l][:HDIM], 4).reshape(1, 128)
        bf2hib = jnp.tile(bf2[l][HDIM:], 4).reshape(1, 128)
        wlo, whi = _filters(dist3, wf1b, bf1b, wf2lob, bf2lob, wf2hib, bf2hib)
        w_all.append((wlo.reshape(E * HDIM), whi.reshape(E * HDIM)))

    xi_prev, xs = _encode(species3, tbl128, W1[0], b1r[0])

    src3 = src.reshape(E // EBLK, 4, EBLK4)
    dst3 = dst.reshape(E // EBLK, 4, EBLK4)
    for l in range(NL):
        agg = _conv(xs.reshape(2 * N, HDIM), w_all[l][0], w_all[l][1], src3, dst3)
        agg3 = agg.reshape(2, N, HDIM)
        if l + 1 < NL:
            xi_prev, xs = _post_mid(agg3, xi_prev, W2[l], b2r[l], W3[l], b3r[l],
                                    W1[l + 1], b1r[l + 1])
        else:
            xi_prev = _post_last(agg3, xi_prev, W2[l], b2r[l], W3[l], b3r[l])
    return xi_prev
